# baseline (speedup 1.0000x reference)
import functools

import jax
import jax.numpy as jnp
from jax import lax
from jax.experimental import pallas as pl
from jax.experimental.pallas import tpu as pltpu
from jax.experimental.pallas import tpu_sc as plsc

B, N, E, BETA = 8, 1024, 2048, 16
HID = 128
NC, NS = 2, 16
NW = NC * NS
EPW = (B * E) // NW
CHUNK = 128
NCHUNK = EPW // CHUNK
RPW = (B * N) // NW

@functools.cache
def _sc_mesh():
    return plsc.VectorSubcoreMesh(core_axis_name="c", subcore_axis_name="s",
                                  num_cores=NC, num_subcores=NS)


@functools.cache
def _efeat_sc_build():
    return pl.kernel(
        _efeat_sc_body,
        mesh=_sc_mesh(),
        out_type=jax.ShapeDtypeStruct((B, 4, E), jnp.float32),
        scratch_types=[
            pltpu.VMEM((2 * N,), jnp.float32),
            pltpu.VMEM((EPW,), jnp.int32),
            pltpu.VMEM((EPW,), jnp.int32),
            pltpu.VMEM((4, EPW), jnp.float32),
        ],
        compiler_params=pltpu.CompilerParams(needs_layout_passes=False),
    )


def _efeat_sc_body(x_hbm, ei_hbm, out_hbm, xv, sv, dv, ev):
    c = lax.axis_index("c")
    s = lax.axis_index("s")
    wid = s * NC + c
    g = wid // (E // EPW)
    q = wid % (E // EPW)
    pltpu.sync_copy(x_hbm.at[g], xv)
    pltpu.sync_copy(ei_hbm.at[g, 0, pl.ds(q * EPW, EPW)], sv)
    pltpu.sync_copy(ei_hbm.at[g, 1, pl.ds(q * EPW, EPW)], dv)

    def body(k, carry):
        sl = pl.ds(k * 16, 16)
        isrc = sv[sl] * 2
        idst = dv[sl] * 2
        ev[0, sl] = plsc.load_gather(xv, [isrc])
        ev[1, sl] = plsc.load_gather(xv, [isrc + 1])
        ev[2, sl] = plsc.load_gather(xv, [idst])
        ev[3, sl] = plsc.load_gather(xv, [idst + 1])
        return carry

    lax.fori_loop(0, EPW // 16, body, 0)
    for i in range(4):
        pltpu.sync_copy(ev.at[i], out_hbm.at[g, i, pl.ds(q * EPW, EPW)])


@functools.cache
def _msg_sc_build(ne=B * E):
    epw = ne // NW
    nchunk = epw // CHUNK

    def _msg_sc_body(h_hbm, src_hbm, out_hbm, siv, gv0, gv1, sem0, sem1, semw):
        c = lax.axis_index("c")
        s = lax.axis_index("s")
        wid = s * NC + c
        base = wid * epw
        pltpu.sync_copy(src_hbm.at[pl.ds(base, epw)], siv)
        bufs = (gv0, gv1)
        sems = (sem0, sem1)
        copies = [None] * nchunk
        copies[0] = pltpu.async_copy(h_hbm.at[siv.at[pl.ds(0, CHUNK)]], gv0,
                                     sem0)
        writes = []
        waited = 0
        for j in range(nchunk):
            if j + 1 < nchunk:
                if j >= 1:
                    writes[j - 1].wait()
                    waited = j
                copies[j + 1] = pltpu.async_copy(
                    h_hbm.at[siv.at[pl.ds((j + 1) * CHUNK, CHUNK)]],
                    bufs[(j + 1) % 2], sems[(j + 1) % 2])
            copies[j].wait()
            writes.append(pltpu.async_copy(
                bufs[j % 2], out_hbm.at[pl.ds(base + j * CHUNK, CHUNK)], semw))
        for w in writes[waited:]:
            w.wait()

    return pl.kernel(
        _msg_sc_body,
        mesh=_sc_mesh(),
        out_type=jax.ShapeDtypeStruct((ne, HID), jnp.float32),
        scratch_types=[
            pltpu.VMEM((epw,), jnp.int32),
            pltpu.VMEM((CHUNK, HID), jnp.float32),
            pltpu.VMEM((CHUNK, HID), jnp.float32),
            pltpu.SemaphoreType.DMA,
            pltpu.SemaphoreType.DMA,
            pltpu.SemaphoreType.DMA,
        ],
    )


def _h0_body(xt_ref, nw, nb, h0_ref):
    xt = xt_ref[0]
    nw_v = nw[...]
    h0_ref[0] = (xt[0][:, None] * nw_v[0][None, :]
                 + xt[1][:, None] * nw_v[1][None, :] + nb[...])


def _h0_tc(xt, nw, nb):
    return pl.pallas_call(
        _h0_body,
        grid=(B,),
        in_specs=[
            pl.BlockSpec((1, 2, N), lambda b: (b, 0, 0)),
            _full(nw.shape), _full(nb.shape),
        ],
        out_specs=pl.BlockSpec((1, N, HID), lambda b: (b, 0, 0)),
        out_shape=jax.ShapeDtypeStruct((B, N, HID), jnp.float32),
    )(xt, nw, nb)


def _dense_body(scb_ref, eft_ref,
                e1w1, e1b1, e1w2, e1b2, w1a, w1b, e2b1, e2w2, e2b2,
                e4w1, e4b1, e4w2, e4b2,
                ea_ref):
    scb_abs = jnp.abs(scb_ref[0])
    eft = eft_ref[0]
    cols = [jnp.sum(scb_abs * eft[i][None, :], axis=1, keepdims=True)
            for i in range(4)]
    sm = jnp.concatenate(cols, axis=1)
    t1 = jnp.maximum(jnp.dot(sm, e1w1[...], preferred_element_type=jnp.float32)
                     + e1b1[...], 0.0)
    h1 = jnp.dot(t1, e1w2[...], preferred_element_type=jnp.float32) + e1b2[...]
    base = jnp.dot(h1, w1a[...], preferred_element_type=jnp.float32) + e2b1[...]
    w1b_v = w1b[...]
    efp = (eft[0][:, None] * w1b_v[0][None, :]
           + eft[1][:, None] * w1b_v[1][None, :]
           + eft[2][:, None] * w1b_v[2][None, :]
           + eft[3][:, None] * w1b_v[3][None, :])
    acc = jnp.zeros((E, HID), jnp.float32)
    for k in range(BETA):
        pre = base[k][None, :] + scb_abs[k][:, None] * efp
        acc = acc + jnp.maximum(pre, 0.0)
    emb = jnp.dot(acc, e2w2[...], preferred_element_type=jnp.float32) \
        + BETA * e2b2[...]
    t4 = jnp.maximum(jnp.dot(emb, e4w1[...], preferred_element_type=jnp.float32)
                     + e4b1[...], 0.0)
    ea_ref[0] = jnp.dot(t4, e4w2[...], preferred_element_type=jnp.float32) \
        + e4b2[...]


def _full(shape):
    nd = len(shape)
    return pl.BlockSpec(shape, lambda b, _nd=nd: (0,) * _nd)


def _dense_tc(scb, eft, e1w1, e1b1, e1w2, e1b2, w1a, w1b, e2b1,
              e2w2, e2b2, e4w1, e4b1, e4w2, e4b2):
    return pl.pallas_call(
        _dense_body,
        grid=(B,),
        in_specs=[
            pl.BlockSpec((1, BETA, E), lambda b: (b, 0, 0)),
            pl.BlockSpec((1, 4, E), lambda b: (b, 0, 0)),
            _full(e1w1.shape), _full(e1b1.shape), _full(e1w2.shape),
            _full(e1b2.shape), _full(w1a.shape), _full(w1b.shape),
            _full(e2b1.shape), _full(e2w2.shape), _full(e2b2.shape),
            _full(e4w1.shape), _full(e4b1.shape), _full(e4w2.shape),
            _full(e4b2.shape),
        ],
        out_specs=pl.BlockSpec((1, E, HID), lambda b: (b, 0, 0)),
        out_shape=jax.ShapeDtypeStruct((B, E, HID), jnp.float32),
    )(scb, eft, e1w1, e1b1, e1w2, e1b2, w1a, w1b, e2b1, e2w2,
      e2b2, e4w1, e4b1, e4w2, e4b2)


_KT = 256


def _seg_sum(gath, ea, dstv):
    agg = jnp.zeros((N, HID), jnp.float32)
    iota_n = lax.broadcasted_iota(jnp.int32, (_KT, N), 1)
    for k in range(E // _KT):
        dk = dstv[k * _KT:(k + 1) * _KT]
        m = (dk[:, None] == iota_n).astype(jnp.bfloat16)
        msg = jnp.maximum(gath[k * _KT:(k + 1) * _KT, :]
                          + ea[k * _KT:(k + 1) * _KT, :], 0.0)
        agg = agg + lax.dot_general(
            m, msg.astype(jnp.bfloat16), (((0,), (0,)), ((), ())),
            preferred_element_type=jnp.float32)
    return agg


def _layer_body(h_ref, msg_ref, ea_ref, dst_ref, w1, b1, w2, b2, out_ref):
    agg = _seg_sum(msg_ref[0], ea_ref[0], dst_ref[0, 0])
    z = h_ref[...] + agg
    t = jnp.maximum(jnp.dot(z, w1[...], preferred_element_type=jnp.float32)
                    + b1[...], 0.0)
    out_ref[...] = jnp.dot(t, w2[...], preferred_element_type=jnp.float32) \
        + b2[...]


def _layer_tc(h, msg, ea, dstl, w1, b1, w2, b2):
    nb = msg.shape[0]
    return pl.pallas_call(
        _layer_body,
        grid=(nb,),
        in_specs=[
            pl.BlockSpec((N, HID), lambda b: (b, 0)),
            pl.BlockSpec((1, E, HID), lambda b: (b, 0, 0)),
            pl.BlockSpec((1, E, HID), lambda b: (b, 0, 0)),
            pl.BlockSpec((1, 1, E), lambda b: (b, 0, 0)),
            _full(w1.shape), _full(b1.shape), _full(w2.shape), _full(b2.shape),
        ],
        out_specs=pl.BlockSpec((N, HID), lambda b: (b, 0)),
        out_shape=jax.ShapeDtypeStruct((nb * N, HID), jnp.float32),
    )(h, msg, ea, dstl, w1, b1, w2, b2)


def _final_body(h_ref, msg_ref, ea_ref, dst_ref, w1, b1, w2, b2, ow, ob, out_ref):
    agg = _seg_sum(msg_ref[0], ea_ref[0], dst_ref[0, 0])
    z = h_ref[...] + agg
    t = jnp.maximum(jnp.dot(z, w1[...], preferred_element_type=jnp.float32)
                    + b1[...], 0.0)
    h2 = jnp.dot(t, w2[...], preferred_element_type=jnp.float32) + b2[...]
    hm = jnp.mean(h2, axis=0, keepdims=True)
    out_ref[0] = jnp.dot(hm, ow[...], preferred_element_type=jnp.float32) \
        + ob[...]


def _final_tc(h, msg, ea, dstl, w1, b1, w2, b2, ow, ob):
    nb = msg.shape[0]
    return pl.pallas_call(
        _final_body,
        grid=(nb,),
        in_specs=[
            pl.BlockSpec((N, HID), lambda b: (b, 0)),
            pl.BlockSpec((1, E, HID), lambda b: (b, 0, 0)),
            pl.BlockSpec((1, E, HID), lambda b: (b, 0, 0)),
            pl.BlockSpec((1, 1, E), lambda b: (b, 0, 0)),
            _full(w1.shape), _full(b1.shape), _full(w2.shape), _full(b2.shape),
            _full(ow.shape), _full(ob.shape),
        ],
        out_specs=pl.BlockSpec((1, 1, HID), lambda b: (b, 0, 0)),
        out_shape=jax.ShapeDtypeStruct((nb, 1, HID), jnp.float32),
    )(h, msg, ea, dstl, w1, b1, w2, b2, ow, ob)


def kernel(x, edge_index, scb, enc1_W1, enc1_b1, enc1_W2, enc1_b2,
           enc2_W1, enc2_b1, enc2_W2, enc2_b2, enc4_W1, enc4_b1, enc4_W2,
           enc4_b2, node_W, node_b, gnn_W1, gnn_b1, gnn_W2, gnn_b2,
           out_W, out_b):
    ei = edge_index.astype(jnp.int32)
    eft = _efeat_sc_build()(x.reshape(B, 2 * N), ei)
    offs = (jnp.arange(B, dtype=jnp.int32) * N)[:, None]
    src_f = (ei[:, 0, :] + offs).reshape(-1)
    dstl = ei[:, 1, :].reshape(B, 1, E)

    xt = x.transpose(0, 2, 1)
    h0 = _h0_tc(xt, node_W, node_b.reshape(1, -1))
    BH = B // 2
    msg_half = _msg_sc_build(BH * E)
    hA = h0[:BH].reshape(BH * N, HID)
    hB = h0[BH:].reshape(BH * N, HID)
    srcA = src_f[:BH * E]
    srcB = src_f[BH * E:] - BH * N
    gA = msg_half(hA, srcA)
    gB = msg_half(hB, srcB)
    ea = _dense_tc(
        scb, eft,
        enc1_W1, enc1_b1.reshape(1, -1), enc1_W2, enc1_b2.reshape(1, -1),
        enc2_W1[:64], enc2_W1[64:], enc2_b1.reshape(1, -1),
        enc2_W2, enc2_b2.reshape(1, -1),
        enc4_W1, enc4_b1.reshape(1, -1), enc4_W2, enc4_b2.reshape(1, -1))
    eaA, eaB = ea[:BH], ea[BH:]
    dstA, dstB = dstl[:BH], dstl[BH:]

    for l in range(2):
        w1, b1 = gnn_W1[l], gnn_b1[l].reshape(1, -1)
        w2, b2 = gnn_W2[l], gnn_b2[l].reshape(1, -1)
        hA = _layer_tc(hA, gA.reshape(BH, E, HID), eaA, dstA, w1, b1, w2, b2)
        gA = msg_half(hA, srcA)
        hB = _layer_tc(hB, gB.reshape(BH, E, HID), eaB, dstB, w1, b1, w2, b2)
        gB = msg_half(hB, srcB)
    w1, b1 = gnn_W1[2], gnn_b1[2].reshape(1, -1)
    w2, b2 = gnn_W2[2], gnn_b2[2].reshape(1, -1)
    ob = out_b.reshape(1, -1)
    outA = _final_tc(hA, gA.reshape(BH, E, HID), eaA, dstA, w1, b1, w2, b2,
                     out_W, ob)
    outB = _final_tc(hB, gB.reshape(BH, E, HID), eaB, dstB, w1, b1, w2, b2,
                     out_W, ob)
    return jnp.concatenate([outA, outB], axis=0).reshape(B, HID)


# speedup vs baseline: 1.0573x; 1.0573x over previous
import functools

import jax
import jax.numpy as jnp
from jax import lax
from jax.experimental import pallas as pl
from jax.experimental.pallas import tpu as pltpu
from jax.experimental.pallas import tpu_sc as plsc

B, N, E, BETA = 8, 1024, 2048, 16
HID = 128
NC, NS = 2, 16
NW = NC * NS
EPW = (B * E) // NW
CHUNK = 128
NCHUNK = EPW // CHUNK
RPW = (B * N) // NW

@functools.cache
def _sc_mesh():
    return plsc.VectorSubcoreMesh(core_axis_name="c", subcore_axis_name="s",
                                  num_cores=NC, num_subcores=NS)


@functools.cache
def _efeat_sc_build():
    return pl.kernel(
        _efeat_sc_body,
        mesh=_sc_mesh(),
        out_type=jax.ShapeDtypeStruct((B, 4, E), jnp.float32),
        scratch_types=[
            pltpu.VMEM((2 * N,), jnp.float32),
            pltpu.VMEM((EPW,), jnp.int32),
            pltpu.VMEM((EPW,), jnp.int32),
            pltpu.VMEM((4, EPW), jnp.float32),
        ],
        compiler_params=pltpu.CompilerParams(needs_layout_passes=False),
    )


def _efeat_sc_body(x_hbm, ei_hbm, out_hbm, xv, sv, dv, ev):
    c = lax.axis_index("c")
    s = lax.axis_index("s")
    wid = s * NC + c
    g = wid // (E // EPW)
    q = wid % (E // EPW)
    pltpu.sync_copy(x_hbm.at[g], xv)
    pltpu.sync_copy(ei_hbm.at[g, 0, pl.ds(q * EPW, EPW)], sv)
    pltpu.sync_copy(ei_hbm.at[g, 1, pl.ds(q * EPW, EPW)], dv)

    def body(k, carry):
        sl = pl.ds(k * 16, 16)
        isrc = sv[sl] * 2
        idst = dv[sl] * 2
        ev[0, sl] = plsc.load_gather(xv, [isrc])
        ev[1, sl] = plsc.load_gather(xv, [isrc + 1])
        ev[2, sl] = plsc.load_gather(xv, [idst])
        ev[3, sl] = plsc.load_gather(xv, [idst + 1])
        return carry

    lax.fori_loop(0, EPW // 16, body, 0)
    for i in range(4):
        pltpu.sync_copy(ev.at[i], out_hbm.at[g, i, pl.ds(q * EPW, EPW)])


@functools.cache
def _msg_sc_build():
    return pl.kernel(
        _msg_sc_body,
        mesh=_sc_mesh(),
        out_type=jax.ShapeDtypeStruct((B * E, HID), jnp.float32),
        scratch_types=[
            pltpu.VMEM((EPW,), jnp.int32),
            pltpu.VMEM((CHUNK, HID), jnp.float32),
            pltpu.VMEM((CHUNK, HID), jnp.float32),
            pltpu.SemaphoreType.DMA,
            pltpu.SemaphoreType.DMA,
            pltpu.SemaphoreType.DMA,
        ],
    )


def _msg_sc_body(h_hbm, src_hbm, out_hbm, siv, gv0, gv1, sem0, sem1, semw):
    c = lax.axis_index("c")
    s = lax.axis_index("s")
    wid = s * NC + c
    base = wid * EPW
    pltpu.sync_copy(src_hbm.at[pl.ds(base, EPW)], siv)
    bufs = (gv0, gv1)
    sems = (sem0, sem1)
    copies = [None] * NCHUNK
    copies[0] = pltpu.async_copy(h_hbm.at[siv.at[pl.ds(0, CHUNK)]], gv0, sem0)
    writes = []
    waited = 0
    for j in range(NCHUNK):
        if j + 1 < NCHUNK:
            if j >= 1:
                writes[j - 1].wait()
                waited = j
            copies[j + 1] = pltpu.async_copy(
                h_hbm.at[siv.at[pl.ds((j + 1) * CHUNK, CHUNK)]],
                bufs[(j + 1) % 2], sems[(j + 1) % 2])
        copies[j].wait()
        writes.append(pltpu.async_copy(
            bufs[j % 2], out_hbm.at[pl.ds(base + j * CHUNK, CHUNK)], semw))
    for w in writes[waited:]:
        w.wait()


def _h0_body(xt_ref, nw, nb, h0_ref):
    xt = xt_ref[0]
    nw_v = nw[...]
    h0_ref[0] = (xt[0][:, None] * nw_v[0][None, :]
                 + xt[1][:, None] * nw_v[1][None, :] + nb[...])


def _h0_tc(xt, nw, nb):
    return pl.pallas_call(
        _h0_body,
        grid=(B,),
        in_specs=[
            pl.BlockSpec((1, 2, N), lambda b: (b, 0, 0)),
            _full(nw.shape), _full(nb.shape),
        ],
        out_specs=pl.BlockSpec((1, N, HID), lambda b: (b, 0, 0)),
        out_shape=jax.ShapeDtypeStruct((B, N, HID), jnp.float32),
    )(xt, nw, nb)


def _dense_body(scb_ref, eft_ref,
                e1w1, e1b1, e1w2, e1b2, w1a, w1b, e2b1, e2w2, e2b2,
                e4w1, e4b1, e4w2, e4b2,
                ea_ref):
    scb_abs = jnp.abs(scb_ref[0])
    eft = eft_ref[0]
    cols = [jnp.sum(scb_abs * eft[i][None, :], axis=1, keepdims=True)
            for i in range(4)]
    sm = jnp.concatenate(cols, axis=1)
    t1 = jnp.maximum(jnp.dot(sm, e1w1[...], preferred_element_type=jnp.float32)
                     + e1b1[...], 0.0)
    h1 = jnp.dot(t1, e1w2[...], preferred_element_type=jnp.float32) + e1b2[...]
    base = jnp.dot(h1, w1a[...], preferred_element_type=jnp.float32) + e2b1[...]
    w1b_v = w1b[...]
    efp = (eft[0][:, None] * w1b_v[0][None, :]
           + eft[1][:, None] * w1b_v[1][None, :]
           + eft[2][:, None] * w1b_v[2][None, :]
           + eft[3][:, None] * w1b_v[3][None, :])
    acc = jnp.zeros((E, HID), jnp.float32)
    for k in range(BETA):
        pre = base[k][None, :] + scb_abs[k][:, None] * efp
        acc = acc + jnp.maximum(pre, 0.0)
    emb = jnp.dot(acc, e2w2[...], preferred_element_type=jnp.float32) \
        + BETA * e2b2[...]
    t4 = jnp.maximum(jnp.dot(emb, e4w1[...], preferred_element_type=jnp.float32)
                     + e4b1[...], 0.0)
    ea_ref[0] = jnp.dot(t4, e4w2[...], preferred_element_type=jnp.float32) \
        + e4b2[...]


def _full(shape):
    nd = len(shape)
    return pl.BlockSpec(shape, lambda b, _nd=nd: (0,) * _nd)


def _dense_tc(scb, eft, e1w1, e1b1, e1w2, e1b2, w1a, w1b, e2b1,
              e2w2, e2b2, e4w1, e4b1, e4w2, e4b2):
    return pl.pallas_call(
        _dense_body,
        grid=(B,),
        in_specs=[
            pl.BlockSpec((1, BETA, E), lambda b: (b, 0, 0)),
            pl.BlockSpec((1, 4, E), lambda b: (b, 0, 0)),
            _full(e1w1.shape), _full(e1b1.shape), _full(e1w2.shape),
            _full(e1b2.shape), _full(w1a.shape), _full(w1b.shape),
            _full(e2b1.shape), _full(e2w2.shape), _full(e2b2.shape),
            _full(e4w1.shape), _full(e4b1.shape), _full(e4w2.shape),
            _full(e4b2.shape),
        ],
        out_specs=pl.BlockSpec((1, E, HID), lambda b: (b, 0, 0)),
        out_shape=jax.ShapeDtypeStruct((B, E, HID), jnp.float32),
    )(scb, eft, e1w1, e1b1, e1w2, e1b2, w1a, w1b, e2b1, e2w2,
      e2b2, e4w1, e4b1, e4w2, e4b2)


_KT = 256


def _seg_sum(gath, ea, dstv):
    agg = jnp.zeros((N, HID), jnp.float32)
    iota_n = lax.broadcasted_iota(jnp.int32, (_KT, N), 1)
    for k in range(E // _KT):
        dk = dstv[k * _KT:(k + 1) * _KT]
        m = (dk[:, None] == iota_n).astype(jnp.bfloat16)
        msg = jnp.maximum(gath[k * _KT:(k + 1) * _KT, :]
                          + ea[k * _KT:(k + 1) * _KT, :], 0.0)
        agg = agg + lax.dot_general(
            m, msg.astype(jnp.bfloat16), (((0,), (0,)), ((), ())),
            preferred_element_type=jnp.float32)
    return agg


def _layer_body(h_ref, msg_ref, ea_ref, dst_ref, w1, b1, w2, b2, out_ref):
    agg = _seg_sum(msg_ref[0], ea_ref[0], dst_ref[0, 0])
    z = h_ref[...] + agg
    t = jnp.maximum(jnp.dot(z, w1[...], preferred_element_type=jnp.float32)
                    + b1[...], 0.0)
    out_ref[...] = jnp.dot(t, w2[...], preferred_element_type=jnp.float32) \
        + b2[...]


def _layer_tc(h, msg, ea, dstl, w1, b1, w2, b2):
    return pl.pallas_call(
        _layer_body,
        grid=(B,),
        in_specs=[
            pl.BlockSpec((N, HID), lambda b: (b, 0)),
            pl.BlockSpec((1, E, HID), lambda b: (b, 0, 0)),
            pl.BlockSpec((1, E, HID), lambda b: (b, 0, 0)),
            pl.BlockSpec((1, 1, E), lambda b: (b, 0, 0)),
            _full(w1.shape), _full(b1.shape), _full(w2.shape), _full(b2.shape),
        ],
        out_specs=pl.BlockSpec((N, HID), lambda b: (b, 0)),
        out_shape=jax.ShapeDtypeStruct((B * N, HID), jnp.float32),
    )(h, msg, ea, dstl, w1, b1, w2, b2)


def _final_body(h_ref, msg_ref, ea_ref, dst_ref, w1, b1, w2, b2, ow, ob, out_ref):
    agg = _seg_sum(msg_ref[0], ea_ref[0], dst_ref[0, 0])
    z = h_ref[...] + agg
    t = jnp.maximum(jnp.dot(z, w1[...], preferred_element_type=jnp.float32)
                    + b1[...], 0.0)
    h2 = jnp.dot(t, w2[...], preferred_element_type=jnp.float32) + b2[...]
    hm = jnp.mean(h2, axis=0, keepdims=True)
    out_ref[0] = jnp.dot(hm, ow[...], preferred_element_type=jnp.float32) \
        + ob[...]


def _final_tc(h, msg, ea, dstl, w1, b1, w2, b2, ow, ob):
    return pl.pallas_call(
        _final_body,
        grid=(B,),
        in_specs=[
            pl.BlockSpec((N, HID), lambda b: (b, 0)),
            pl.BlockSpec((1, E, HID), lambda b: (b, 0, 0)),
            pl.BlockSpec((1, E, HID), lambda b: (b, 0, 0)),
            pl.BlockSpec((1, 1, E), lambda b: (b, 0, 0)),
            _full(w1.shape), _full(b1.shape), _full(w2.shape), _full(b2.shape),
            _full(ow.shape), _full(ob.shape),
        ],
        out_specs=pl.BlockSpec((1, 1, HID), lambda b: (b, 0, 0)),
        out_shape=jax.ShapeDtypeStruct((B, 1, HID), jnp.float32),
    )(h, msg, ea, dstl, w1, b1, w2, b2, ow, ob)


def kernel(x, edge_index, scb, enc1_W1, enc1_b1, enc1_W2, enc1_b2,
           enc2_W1, enc2_b1, enc2_W2, enc2_b2, enc4_W1, enc4_b1, enc4_W2,
           enc4_b2, node_W, node_b, gnn_W1, gnn_b1, gnn_W2, gnn_b2,
           out_W, out_b):
    ei = edge_index.astype(jnp.int32)
    eft = _efeat_sc_build()(x.reshape(B, 2 * N), ei)
    offs = (jnp.arange(B, dtype=jnp.int32) * N)[:, None]
    src_f = (ei[:, 0, :] + offs).reshape(-1)
    dstl = ei[:, 1, :].reshape(B, 1, E)

    xt = x.transpose(0, 2, 1)
    h = _h0_tc(xt, node_W, node_b.reshape(1, -1)).reshape(B * N, HID)
    gath = _msg_sc_build()(h, src_f)
    ea = _dense_tc(
        scb, eft,
        enc1_W1, enc1_b1.reshape(1, -1), enc1_W2, enc1_b2.reshape(1, -1),
        enc2_W1[:64], enc2_W1[64:], enc2_b1.reshape(1, -1),
        enc2_W2, enc2_b2.reshape(1, -1),
        enc4_W1, enc4_b1.reshape(1, -1), enc4_W2, enc4_b2.reshape(1, -1))

    for l in range(2):
        h = _layer_tc(h, gath.reshape(B, E, HID), ea, dstl,
                      gnn_W1[l], gnn_b1[l].reshape(1, -1),
                      gnn_W2[l], gnn_b2[l].reshape(1, -1))
        gath = _msg_sc_build()(h, src_f)
    out = _final_tc(h, gath.reshape(B, E, HID), ea, dstl,
                    gnn_W1[2], gnn_b1[2].reshape(1, -1),
                    gnn_W2[2], gnn_b2[2].reshape(1, -1),
                    out_W, out_b.reshape(1, -1))
    return out.reshape(B, HID)
